# baseline (speedup 1.0000x reference)
import functools

import jax
import jax.numpy as jnp
from jax import lax
from jax.experimental import pallas as pl
from jax.experimental.pallas import tpu as pltpu
from jax.experimental.pallas import tpu_sc as plsc

B = 4096
L = 200
EMB_DIM = 64
HIDDEN = 256
OUT = 5

NC, NS = 2, 16
NW = NC * NS
ROWS_PER_W = B // NW
CHUNK = 104
LP = 2 * CHUNK
CHUNKS_PER_ROW = 2
CHUNKS_PER_W = ROWS_PER_W * CHUNKS_PER_ROW
NCHUNKS = B * CHUNKS_PER_ROW
NBUF = 4
NGRP = CHUNKS_PER_W // NBUF
NJ = EMB_DIM // 16

OUT_PAD = 128


def _pool_body(idx_hbm, dst_hbm, emb_hbm, out_hbm,
               idx_v, dst_v, b0, b1, b2, b3, zbuf, shared,
               g0, g1, g2, g3, s0, s1, s2, s3):
    bufs = (b0, b1, b2, b3)
    gsems = (g0, g1, g2, g3)
    ssems = (s0, s1, s2, s3)
    sid = lax.axis_index("s")
    wid = sid * NC + lax.axis_index("c")
    cbase = wid * CHUNKS_PER_W
    pltpu.sync_copy(idx_hbm.at[pl.ds(cbase, CHUNKS_PER_W)], idx_v)
    pltpu.sync_copy(dst_hbm.at[pl.ds(cbase, CHUNKS_PER_W)], dst_v)

    zero = jnp.zeros((16,), jnp.float32)

    def zero_body(i, carry):
        for j in range(NJ):
            zbuf[i, pl.ds(16 * j, 16)] = zero
        return carry

    lax.fori_loop(0, ROWS_PER_W, zero_body, 0)
    pltpu.sync_copy(zbuf, shared.at[pl.ds(sid * ROWS_PER_W, ROWS_PER_W)])

    for b in range(NBUF):
        pltpu.async_copy(emb_hbm.at[idx_v.at[b]], bufs[b], gsems[b])

    def grp_body(g, carry):
        c0 = NBUF * g
        for b in range(NBUF):
            pltpu.make_async_copy(
                emb_hbm.at[idx_v.at[0]], bufs[b], gsems[b]).wait()
            pltpu.async_copy(
                bufs[b], shared.at[dst_v.at[c0 + b]], ssems[b], add=True)
        for b in range(NBUF):
            pltpu.make_async_copy(
                bufs[b], shared.at[dst_v.at[0]], ssems[b]).wait()

            @pl.when(g < NGRP - 1)
            def _():
                pltpu.async_copy(
                    emb_hbm.at[idx_v.at[c0 + b + NBUF]], bufs[b], gsems[b])
        return carry

    lax.fori_loop(0, NGRP, grp_body, 0)
    pltpu.sync_copy(shared.at[pl.ds(sid * ROWS_PER_W, ROWS_PER_W)],
                    out_hbm.at[pl.ds(wid * ROWS_PER_W, ROWS_PER_W)])


@functools.lru_cache(maxsize=1)
def _make_pool():
    return pl.kernel(
        _pool_body,
        out_type=jax.ShapeDtypeStruct((B, EMB_DIM), jnp.float32),
        mesh=plsc.VectorSubcoreMesh(core_axis_name="c", subcore_axis_name="s"),
        compiler_params=pltpu.CompilerParams(use_tc_tiling_on_sc=False),
        scratch_types=(
            [pltpu.VMEM((CHUNKS_PER_W, CHUNK), jnp.int32),
             pltpu.VMEM((CHUNKS_PER_W, CHUNK), jnp.int32)]
            + [pltpu.VMEM((CHUNK, EMB_DIM), jnp.float32)
               for _ in range(NBUF)]
            + [pltpu.VMEM((ROWS_PER_W, EMB_DIM), jnp.float32),
               pltpu.VMEM_SHARED((NS * ROWS_PER_W, EMB_DIM), jnp.float32)]
            + [pltpu.SemaphoreType.DMA for _ in range(2 * NBUF)]
        ),
    )


def _mlp_body(x_ref, w1t_ref, b1_ref, w2t_ref, b2_ref, o_ref):
    x = x_ref[:] * (1.0 / L)
    h = jnp.dot(x, w1t_ref[:], preferred_element_type=jnp.float32) + b1_ref[:]
    h = jnp.maximum(h, 0.0)
    o = jnp.dot(h, w2t_ref[:], preferred_element_type=jnp.float32) + b2_ref[:]
    m = jnp.max(o, axis=1, keepdims=True)
    lse = jnp.log(jnp.sum(jnp.exp(o - m), axis=1, keepdims=True)) + m
    o_ref[:] = o - lse


def _mlp(sums, w1t, b1_2d, w2tp, b2p):
    blk = B // 4
    return pl.pallas_call(
        _mlp_body,
        grid=(4,),
        in_specs=[
            pl.BlockSpec((blk, EMB_DIM), lambda i: (i, 0)),
            pl.BlockSpec((EMB_DIM, HIDDEN), lambda i: (0, 0)),
            pl.BlockSpec((1, HIDDEN), lambda i: (0, 0)),
            pl.BlockSpec((HIDDEN, OUT_PAD), lambda i: (0, 0)),
            pl.BlockSpec((1, OUT_PAD), lambda i: (0, 0)),
        ],
        out_specs=pl.BlockSpec((blk, OUT_PAD), lambda i: (i, 0)),
        out_shape=jax.ShapeDtypeStruct((B, OUT_PAD), jnp.float32),
    )(sums, w1t, b1_2d, w2tp, b2p)


def kernel(subword_indices, emb, W1, b1, W2, b2):
    idx = subword_indices.astype(jnp.int32)
    idx = jnp.pad(idx, ((0, 0), (0, LP - L)))
    idx = idx.reshape(NCHUNKS, CHUNK)
    c = jnp.arange(NCHUNKS, dtype=jnp.int32)
    dstval = (c // (2 * CHUNKS_PER_W)) * ROWS_PER_W + (c % CHUNKS_PER_W) // 2
    dst = jnp.broadcast_to(dstval[:, None], (NCHUNKS, CHUNK))
    sums = _make_pool()(idx, dst, emb)
    w1t = W1.T
    b1_2d = b1.reshape(1, HIDDEN)
    w2tp = jnp.zeros((HIDDEN, OUT_PAD), jnp.float32).at[:, :OUT].set(W2.T)
    b2p = jnp.full((1, OUT_PAD), -1e30, jnp.float32).at[0, :OUT].set(b2)
    out = _mlp(sums, w1t, b1_2d, w2tp, b2p)
    return out[:, :OUT]


# speedup vs baseline: 4.2587x; 4.2587x over previous
import functools

import jax
import jax.numpy as jnp
from jax import lax
from jax.experimental import pallas as pl
from jax.experimental.pallas import tpu as pltpu
from jax.experimental.pallas import tpu_sc as plsc

B = 4096
L = 200
EMB_DIM = 64
HIDDEN = 256
OUT = 5

NC, NS = 2, 16
NW = NC * NS
ROWS_PER_W = B // NW
CHUNK = 100
CHUNKS_PER_ROW = L // CHUNK
CHUNKS_PER_W = ROWS_PER_W * CHUNKS_PER_ROW
NBUF = 4
NGRP = CHUNKS_PER_W // NBUF
HALF = CHUNK // 2
NH = EMB_DIM // 32

OUT_PAD = 128


def _pool_body(idx_hbm, emb_hbm, out_hbm, idx_v, b0, b1, b2, b3,
               acc_v, s0, s1, s2, s3):
    bufs = (b0, b1, b2, b3)
    sems = (s0, s1, s2, s3)
    wid = lax.axis_index("s") * NC + lax.axis_index("c")
    cbase = wid * CHUNKS_PER_W
    pltpu.sync_copy(idx_hbm.at[pl.ds(cbase, CHUNKS_PER_W)], idx_v)

    lanes = lax.iota(jnp.int32, 16)
    cols = [(32 * h + 2 * lanes, 32 * h + 2 * lanes + 1) for h in range(NH)]

    for b in range(NBUF):
        pltpu.async_copy(emb_hbm.at[idx_v.at[b]], bufs[b], sems[b])

    def acc_chunk(buf, chains):
        for t in range(CHUNK):
            for h in range(NH):
                x = buf[t, pl.ds(32 * h, 32)]
                e, o = plsc.unpack(x, format=plsc.PackFormat.INTERLEAVED)
                hv = 0 if t < HALF else 1
                for eo, v in ((0, e), (1, o)):
                    cur = chains[h][eo][hv]
                    chains[h][eo][hv] = v if cur is None else cur + v
        return chains

    def grp_body(g, carry):
        c0 = NBUF * g
        for p in range(NBUF // 2):
            chains = [[[None, None], [None, None]] for _ in range(NH)]
            for q in range(2):
                b = 2 * p + q
                pltpu.make_async_copy(
                    emb_hbm.at[idx_v.at[0]], bufs[b], sems[b]).wait()
                chains = acc_chunk(bufs[b], chains)

                @pl.when(g < NGRP - 1)
                def _():
                    pltpu.async_copy(
                        emb_hbm.at[idx_v.at[c0 + b + NBUF]], bufs[b], sems[b])
            r = (NBUF // 2) * g + p
            rvec = jnp.full((16,), 0, jnp.int32) + r
            for h in range(NH):
                for eo in range(2):
                    tot = chains[h][eo][0] + chains[h][eo][1]
                    plsc.store_scatter(acc_v, [rvec, cols[h][eo]], tot)
        return carry

    lax.fori_loop(0, NGRP, grp_body, 0)
    pltpu.sync_copy(acc_v, out_hbm.at[pl.ds(wid * ROWS_PER_W, ROWS_PER_W)])


@functools.lru_cache(maxsize=1)
def _make_pool():
    return pl.kernel(
        _pool_body,
        out_type=jax.ShapeDtypeStruct((B, EMB_DIM), jnp.float32),
        mesh=plsc.VectorSubcoreMesh(core_axis_name="c", subcore_axis_name="s"),
        compiler_params=pltpu.CompilerParams(
            use_tc_tiling_on_sc=False, needs_layout_passes=False),
        scratch_types=(
            [pltpu.VMEM((CHUNKS_PER_W, CHUNK), jnp.int32)]
            + [pltpu.VMEM((CHUNK, EMB_DIM), jnp.bfloat16)
               for _ in range(NBUF)]
            + [pltpu.VMEM((ROWS_PER_W, EMB_DIM), jnp.float32)]
            + [pltpu.SemaphoreType.DMA for _ in range(NBUF)]
        ),
    )


def _mlp_body(x_ref, w1t_ref, b1_ref, w2t_ref, b2_ref, o_ref):
    x = x_ref[:] * (1.0 / L)
    h = jnp.dot(x, w1t_ref[:], preferred_element_type=jnp.float32) + b1_ref[:]
    h = jnp.maximum(h, 0.0)
    o = jnp.dot(h, w2t_ref[:], preferred_element_type=jnp.float32) + b2_ref[:]
    m = jnp.max(o, axis=1, keepdims=True)
    lse = jnp.log(jnp.sum(jnp.exp(o - m), axis=1, keepdims=True)) + m
    o_ref[:] = o - lse


def _mlp(sums, w1t, b1_2d, w2tp, b2p):
    blk = B // 4
    return pl.pallas_call(
        _mlp_body,
        grid=(4,),
        in_specs=[
            pl.BlockSpec((blk, EMB_DIM), lambda i: (i, 0)),
            pl.BlockSpec((EMB_DIM, HIDDEN), lambda i: (0, 0)),
            pl.BlockSpec((1, HIDDEN), lambda i: (0, 0)),
            pl.BlockSpec((HIDDEN, OUT_PAD), lambda i: (0, 0)),
            pl.BlockSpec((1, OUT_PAD), lambda i: (0, 0)),
        ],
        out_specs=pl.BlockSpec((blk, OUT_PAD), lambda i: (i, 0)),
        out_shape=jax.ShapeDtypeStruct((B, OUT_PAD), jnp.float32),
    )(sums, w1t, b1_2d, w2tp, b2p)


def kernel(subword_indices, emb, W1, b1, W2, b2):
    idx = subword_indices.astype(jnp.int32).reshape(B * L // CHUNK, CHUNK)
    sums = _make_pool()(idx, emb.astype(jnp.bfloat16))
    w1t = W1.T
    b1_2d = b1.reshape(1, HIDDEN)
    w2tp = jnp.zeros((HIDDEN, OUT_PAD), jnp.float32).at[:, :OUT].set(W2.T)
    b2p = jnp.full((1, OUT_PAD), -1e30, jnp.float32).at[0, :OUT].set(b2)
    out = _mlp(sums, w1t, b1_2d, w2tp, b2p)
    return out[:, :OUT]
